# initial kernel scaffold (unmeasured)
import jax
import jax.numpy as jnp
from jax import lax
from jax.experimental import pallas as pl
from jax.experimental.pallas import tpu as pltpu

N_DEV = 8


def kernel(A, B):
    m_per, k = A.shape
    _, n = B.shape

    a16 = A.astype(jnp.bfloat16)
    b16 = B.astype(jnp.bfloat16)

    def body(a_ref, b_ref, out_ref, comm_ref, send_sems, recv_sems,
             copy_sem, capacity_sem):
        my = lax.axis_index("i")
        left = lax.rem(my + N_DEV - 1, N_DEV)
        right = lax.rem(my + 1, N_DEV)

        barrier_sem = pltpu.get_barrier_semaphore()
        for nbr in (left, right):
            pl.semaphore_signal(
                barrier_sem, inc=1,
                device_id=(nbr,), device_id_type=pl.DeviceIdType.MESH,
            )
        pl.semaphore_wait(barrier_sem, 2)

        n_tile = n // 4
        for j in range(4):
            js = pl.ds(j * n_tile, n_tile)
            comm_ref[0, :, js] = jnp.dot(
                a_ref[:, :], b_ref[:, js],
                preferred_element_type=jnp.float32,
            ).astype(jnp.bfloat16)

        copy0 = pltpu.make_async_copy(
            comm_ref.at[0], out_ref.at[pl.ds(my * m_per, m_per), :], copy_sem)
        copy0.start()
        copy0.wait()

        for h in range(N_DEV - 1):
            s_send = h % 2
            s_recv = (h + 1) % 2
            if h >= 1:
                pl.semaphore_wait(capacity_sem, 1)
            rdma = pltpu.make_async_remote_copy(
                src_ref=comm_ref.at[s_send],
                dst_ref=comm_ref.at[s_recv],
                send_sem=send_sems.at[s_send],
                recv_sem=recv_sems.at[s_recv],
                device_id=(right,),
                device_id_type=pl.DeviceIdType.MESH,
            )
            rdma.start()
            rdma.wait()
            if h < N_DEV - 2:
                pl.semaphore_signal(
                    capacity_sem, inc=1,
                    device_id=(left,), device_id_type=pl.DeviceIdType.MESH,
                )
            origin = lax.rem(my - h - 1 + N_DEV, N_DEV)
            cp = pltpu.make_async_copy(
                comm_ref.at[s_recv],
                out_ref.at[pl.ds(origin * m_per, m_per), :], copy_sem)
            cp.start()
            cp.wait()

    return pl.pallas_call(
        body,
        out_shape=jax.ShapeDtypeStruct((N_DEV * m_per, n), jnp.bfloat16),
        in_specs=[
            pl.BlockSpec(memory_space=pltpu.VMEM),
            pl.BlockSpec(memory_space=pltpu.VMEM),
        ],
        out_specs=pl.BlockSpec(memory_space=pltpu.ANY),
        scratch_shapes=[
            pltpu.VMEM((2, m_per, n), jnp.bfloat16),
            pltpu.SemaphoreType.DMA((2,)),
            pltpu.SemaphoreType.DMA((2,)),
            pltpu.SemaphoreType.DMA,
            pltpu.SemaphoreType.REGULAR,
        ],
        compiler_params=pltpu.CompilerParams(collective_id=0),
    )(a16, b16)


# baseline (device time: 1636880 ns/iter reference)
import jax
import jax.numpy as jnp
from jax import lax
from jax.experimental import pallas as pl
from jax.experimental.pallas import tpu as pltpu

N_DEV = 8


def kernel(A, B):
    m_per, k = A.shape
    _, n = B.shape

    a16 = A.astype(jnp.bfloat16)
    b16 = B.astype(jnp.bfloat16)

    def body(a_ref, b_ref, out_ref, comm_ref, send_sems, recv_sems,
             copy_sem, capacity_sem):
        my = lax.axis_index("i")
        left = lax.rem(my + N_DEV - 1, N_DEV)
        right = lax.rem(my + 1, N_DEV)

        barrier_sem = pltpu.get_barrier_semaphore()
        for nbr in (left, right):
            pl.semaphore_signal(
                barrier_sem, inc=1,
                device_id=(nbr,), device_id_type=pl.DeviceIdType.MESH,
            )
        pl.semaphore_wait(barrier_sem, 2)

        n_tile = n // 4
        for j in range(4):
            js = pl.ds(j * n_tile, n_tile)
            comm_ref[0, :, js] = jnp.dot(
                a_ref[:, :], b_ref[:, js],
                preferred_element_type=jnp.float32,
            ).astype(jnp.bfloat16)

        copy0 = pltpu.make_async_copy(
            comm_ref.at[0], out_ref.at[pl.ds(my * m_per, m_per), :], copy_sem)
        copy0.start()
        copy0.wait()

        for h in range(N_DEV - 1):
            s_send = h % 2
            s_recv = (h + 1) % 2
            if h >= 1:
                pl.semaphore_wait(capacity_sem, 1)
            rdma = pltpu.make_async_remote_copy(
                src_ref=comm_ref.at[s_send],
                dst_ref=comm_ref.at[s_recv],
                send_sem=send_sems.at[s_send],
                recv_sem=recv_sems.at[s_recv],
                device_id=(right,),
                device_id_type=pl.DeviceIdType.MESH,
            )
            rdma.start()
            rdma.wait()
            if h < N_DEV - 2:
                pl.semaphore_signal(
                    capacity_sem, inc=1,
                    device_id=(left,), device_id_type=pl.DeviceIdType.MESH,
                )
            origin = lax.rem(my - h - 1 + N_DEV, N_DEV)
            cp = pltpu.make_async_copy(
                comm_ref.at[s_recv],
                out_ref.at[pl.ds(origin * m_per, m_per), :], copy_sem)
            cp.start()
            cp.wait()

    return pl.pallas_call(
        body,
        out_shape=jax.ShapeDtypeStruct((N_DEV * m_per, n), jnp.bfloat16),
        in_specs=[
            pl.BlockSpec(memory_space=pltpu.VMEM),
            pl.BlockSpec(memory_space=pltpu.VMEM),
        ],
        out_specs=pl.BlockSpec(memory_space=pl.ANY),
        scratch_shapes=[
            pltpu.VMEM((2, m_per, n), jnp.bfloat16),
            pltpu.SemaphoreType.DMA((2,)),
            pltpu.SemaphoreType.DMA((2,)),
            pltpu.SemaphoreType.DMA,
            pltpu.SemaphoreType.REGULAR,
        ],
        compiler_params=pltpu.CompilerParams(
            collective_id=0,
            vmem_limit_bytes=100 * 1024 * 1024,
        ),
    )(a16, b16)


# device time: 893572 ns/iter; 1.8318x vs baseline; 1.8318x over previous
import jax
import jax.numpy as jnp
from jax import lax
from jax.experimental import pallas as pl
from jax.experimental.pallas import tpu as pltpu

N_DEV = 8


def kernel(A, B):
    m_per, k = A.shape
    _, n = B.shape
    m_half = m_per // 2

    a16 = A.astype(jnp.bfloat16)
    b16 = B.astype(jnp.bfloat16)

    def body(a_ref, b_ref, out_ref, comm_r, comm_l,
             send_r, recv_r, send_l, recv_l, cp_sems, cap_r, cap_l):
        my = lax.axis_index("i")
        left = lax.rem(my + N_DEV - 1, N_DEV)
        right = lax.rem(my + 1, N_DEV)

        barrier_sem = pltpu.get_barrier_semaphore()
        for nbr in (left, right):
            pl.semaphore_signal(
                barrier_sem, inc=1,
                device_id=(nbr,), device_id_type=pl.DeviceIdType.MESH,
            )
        pl.semaphore_wait(barrier_sem, 2)

        n_tile = n // 2
        for j in range(2):
            js = pl.ds(j * n_tile, n_tile)
            comm_r[0, :, js] = jnp.dot(
                a_ref[pl.ds(0, m_half), :], b_ref[:, js],
                preferred_element_type=jnp.float32,
            ).astype(jnp.bfloat16)
            comm_l[0, :, js] = jnp.dot(
                a_ref[pl.ds(m_half, m_half), :], b_ref[:, js],
                preferred_element_type=jnp.float32,
            ).astype(jnp.bfloat16)

        cp0 = pltpu.make_async_copy(
            comm_r.at[0], out_ref.at[pl.ds(my * m_per, m_half), :],
            cp_sems.at[0])
        cp1 = pltpu.make_async_copy(
            comm_l.at[0], out_ref.at[pl.ds(my * m_per + m_half, m_half), :],
            cp_sems.at[1])
        cp0.start()
        cp1.start()
        pending = [cp0, cp1]

        for h in range(N_DEV - 1):
            ss = h % 2
            sr = (h + 1) % 2
            if h >= 1:
                pl.semaphore_wait(cap_r, 1)
                pl.semaphore_wait(cap_l, 1)
            rd_r = pltpu.make_async_remote_copy(
                src_ref=comm_r.at[ss], dst_ref=comm_r.at[sr],
                send_sem=send_r.at[ss], recv_sem=recv_r.at[sr],
                device_id=(right,), device_id_type=pl.DeviceIdType.MESH,
            )
            rd_l = pltpu.make_async_remote_copy(
                src_ref=comm_l.at[ss], dst_ref=comm_l.at[sr],
                send_sem=send_l.at[ss], recv_sem=recv_l.at[sr],
                device_id=(left,), device_id_type=pl.DeviceIdType.MESH,
            )
            rd_r.start()
            rd_l.start()
            rd_r.wait()
            rd_l.wait()
            for cp in pending:
                cp.wait()
            if h < N_DEV - 2:
                pl.semaphore_signal(
                    cap_r, inc=1,
                    device_id=(left,), device_id_type=pl.DeviceIdType.MESH)
                pl.semaphore_signal(
                    cap_l, inc=1,
                    device_id=(right,), device_id_type=pl.DeviceIdType.MESH)
            orig_r = lax.rem(my - h - 1 + N_DEV, N_DEV)
            orig_l = lax.rem(my + h + 1, N_DEV)
            cp0 = pltpu.make_async_copy(
                comm_r.at[sr],
                out_ref.at[pl.ds(orig_r * m_per, m_half), :], cp_sems.at[0])
            cp1 = pltpu.make_async_copy(
                comm_l.at[sr],
                out_ref.at[pl.ds(orig_l * m_per + m_half, m_half), :],
                cp_sems.at[1])
            cp0.start()
            cp1.start()
            pending = [cp0, cp1]
        for cp in pending:
            cp.wait()

    return pl.pallas_call(
        body,
        out_shape=jax.ShapeDtypeStruct((N_DEV * m_per, n), jnp.bfloat16),
        in_specs=[
            pl.BlockSpec(memory_space=pltpu.VMEM),
            pl.BlockSpec(memory_space=pltpu.VMEM),
        ],
        out_specs=pl.BlockSpec(memory_space=pl.ANY),
        scratch_shapes=[
            pltpu.VMEM((2, m_half, n), jnp.bfloat16),
            pltpu.VMEM((2, m_half, n), jnp.bfloat16),
            pltpu.SemaphoreType.DMA((2,)),
            pltpu.SemaphoreType.DMA((2,)),
            pltpu.SemaphoreType.DMA((2,)),
            pltpu.SemaphoreType.DMA((2,)),
            pltpu.SemaphoreType.DMA((2,)),
            pltpu.SemaphoreType.REGULAR,
            pltpu.SemaphoreType.REGULAR,
        ],
        compiler_params=pltpu.CompilerParams(
            collective_id=0,
            vmem_limit_bytes=100 * 1024 * 1024,
        ),
    )(a16, b16)


# device time: 546942 ns/iter; 2.9928x vs baseline; 1.6338x over previous
import jax
import jax.numpy as jnp
from jax import lax
from jax.experimental import pallas as pl
from jax.experimental.pallas import tpu as pltpu

N_DEV = 8


def kernel(A, B):
    m_per, k = A.shape
    _, n = B.shape
    m_half = m_per // 2

    a16 = A.astype(jnp.bfloat16)
    b16 = B.astype(jnp.bfloat16)

    def body(a_ref, b_ref, out_ref, comm, stage,
             send_r, recv_r, send_l, recv_l, cp_sems, cap_r, cap_l):
        my = lax.axis_index("i")
        left = lax.rem(my + N_DEV - 1, N_DEV)
        right = lax.rem(my + 1, N_DEV)

        barrier_sem = pltpu.get_barrier_semaphore()
        for nbr in (left, right):
            pl.semaphore_signal(
                barrier_sem, inc=1,
                device_id=(nbr,), device_id_type=pl.DeviceIdType.MESH,
            )
        pl.semaphore_wait(barrier_sem, 2)

        seed = pltpu.make_async_copy(a_ref, comm.at[0], cp_sems.at[0])
        seed.start()
        seed.wait()

        def out_copies(orig_r, orig_l):
            cp_t = pltpu.make_async_copy(
                stage.at[pl.ds(0, m_half)],
                out_ref.at[pl.ds(orig_r * m_per, m_half), :],
                cp_sems.at[0])
            cp_b = pltpu.make_async_copy(
                stage.at[pl.ds(m_half, m_half)],
                out_ref.at[pl.ds(orig_l * m_per + m_half, m_half), :],
                cp_sems.at[1])
            return cp_t, cp_b

        def compute(ss, orig_r, orig_l):
            n_tile = n // 4
            for j in range(4):
                js = pl.ds(j * n_tile, n_tile)
                stage[:, js] = jnp.dot(
                    comm[ss], b_ref[:, js],
                    preferred_element_type=jnp.float32,
                ).astype(jnp.bfloat16)
            cp_t, cp_b = out_copies(orig_r, orig_l)
            cp_t.start()
            cp_b.start()

        def hop(h, carry):
            ss = lax.rem(h, 2)
            sr = lax.rem(h + 1, 2)

            @pl.when(h >= 1)
            def _():
                pl.semaphore_wait(cap_r, 1)
                pl.semaphore_wait(cap_l, 1)

            rd_r = pltpu.make_async_remote_copy(
                src_ref=comm.at[ss, pl.ds(0, m_half)],
                dst_ref=comm.at[sr, pl.ds(0, m_half)],
                send_sem=send_r.at[ss], recv_sem=recv_r.at[sr],
                device_id=(right,), device_id_type=pl.DeviceIdType.MESH,
            )
            rd_l = pltpu.make_async_remote_copy(
                src_ref=comm.at[ss, pl.ds(m_half, m_half)],
                dst_ref=comm.at[sr, pl.ds(m_half, m_half)],
                send_sem=send_l.at[ss], recv_sem=recv_l.at[sr],
                device_id=(left,), device_id_type=pl.DeviceIdType.MESH,
            )
            rd_r.start()
            rd_l.start()

            @pl.when(h >= 1)
            def _():
                pcp_t, pcp_b = out_copies(
                    lax.rem(my - (h - 1) + N_DEV, N_DEV),
                    lax.rem(my + (h - 1), N_DEV))
                pcp_t.wait()
                pcp_b.wait()

            compute(ss,
                    lax.rem(my - h + N_DEV, N_DEV),
                    lax.rem(my + h, N_DEV))

            rd_r.wait()
            rd_l.wait()

            @pl.when(h < N_DEV - 2)
            def _():
                pl.semaphore_signal(
                    cap_r, inc=1,
                    device_id=(left,), device_id_type=pl.DeviceIdType.MESH)
                pl.semaphore_signal(
                    cap_l, inc=1,
                    device_id=(right,), device_id_type=pl.DeviceIdType.MESH)

            return carry

        lax.fori_loop(0, N_DEV - 1, hop, 0)

        lcp_t, lcp_b = out_copies(
            lax.rem(my - (N_DEV - 2) + N_DEV, N_DEV),
            lax.rem(my + (N_DEV - 2), N_DEV))
        lcp_t.wait()
        lcp_b.wait()
        compute(1,
                lax.rem(my + 1, N_DEV),
                lax.rem(my + N_DEV - 1, N_DEV))
        fcp_t, fcp_b = out_copies(
            lax.rem(my + 1, N_DEV), lax.rem(my + N_DEV - 1, N_DEV))
        fcp_t.wait()
        fcp_b.wait()

    return pl.pallas_call(
        body,
        out_shape=jax.ShapeDtypeStruct((N_DEV * m_per, n), jnp.bfloat16),
        in_specs=[
            pl.BlockSpec(memory_space=pl.ANY),
            pl.BlockSpec(memory_space=pltpu.VMEM),
        ],
        out_specs=pl.BlockSpec(memory_space=pl.ANY),
        scratch_shapes=[
            pltpu.VMEM((2, m_per, k), jnp.bfloat16),
            pltpu.VMEM((m_per, n), jnp.bfloat16),
            pltpu.SemaphoreType.DMA((2,)),
            pltpu.SemaphoreType.DMA((2,)),
            pltpu.SemaphoreType.DMA((2,)),
            pltpu.SemaphoreType.DMA((2,)),
            pltpu.SemaphoreType.DMA((2,)),
            pltpu.SemaphoreType.REGULAR,
            pltpu.SemaphoreType.REGULAR,
        ],
        compiler_params=pltpu.CompilerParams(
            collective_id=0,
            vmem_limit_bytes=100 * 1024 * 1024,
        ),
    )(a16, b16)


# device time: 540749 ns/iter; 3.0271x vs baseline; 1.0115x over previous
import jax
import jax.numpy as jnp
from jax import lax
from jax.experimental import pallas as pl
from jax.experimental.pallas import tpu as pltpu

N_DEV = 8


def kernel(A, B):
    m_per, k = A.shape
    _, n = B.shape
    m_half = m_per // 2

    a16 = A.astype(jnp.bfloat16)
    b16 = B.astype(jnp.bfloat16)

    def body(a_ref, b_ref, out_ref, comm, stage,
             send_r, recv_r, send_l, recv_l, cp_sems, cap_r, cap_l):
        my = lax.axis_index("i")
        left = lax.rem(my + N_DEV - 1, N_DEV)
        right = lax.rem(my + 1, N_DEV)

        seed = pltpu.make_async_copy(a_ref, comm.at[0], cp_sems.at[0])
        seed.start()

        barrier_sem = pltpu.get_barrier_semaphore()
        for nbr in (left, right):
            pl.semaphore_signal(
                barrier_sem, inc=1,
                device_id=(nbr,), device_id_type=pl.DeviceIdType.MESH,
            )
        pl.semaphore_wait(barrier_sem, 2)
        seed.wait()

        def out_copies(orig_r, orig_l):
            cp_t = pltpu.make_async_copy(
                stage.at[pl.ds(0, m_half)],
                out_ref.at[pl.ds(orig_r * m_per, m_half), :],
                cp_sems.at[0])
            cp_b = pltpu.make_async_copy(
                stage.at[pl.ds(m_half, m_half)],
                out_ref.at[pl.ds(orig_l * m_per + m_half, m_half), :],
                cp_sems.at[1])
            return cp_t, cp_b

        def compute(ss, orig_r, orig_l):
            n_tile = n // 4
            for j in range(4):
                js = pl.ds(j * n_tile, n_tile)
                stage[:, js] = jnp.dot(
                    comm[ss], b_ref[:, js],
                    preferred_element_type=jnp.float32,
                ).astype(jnp.bfloat16)
            cp_t, cp_b = out_copies(orig_r, orig_l)
            cp_t.start()
            cp_b.start()

        def hop(h, carry):
            ss = lax.rem(h, 2)
            sr = lax.rem(h + 1, 2)

            @pl.when(h >= 1)
            def _():
                pl.semaphore_wait(cap_r, 1)
                pl.semaphore_wait(cap_l, 1)

            rd_r = pltpu.make_async_remote_copy(
                src_ref=comm.at[ss, pl.ds(0, m_half)],
                dst_ref=comm.at[sr, pl.ds(0, m_half)],
                send_sem=send_r.at[ss], recv_sem=recv_r.at[sr],
                device_id=(right,), device_id_type=pl.DeviceIdType.MESH,
            )
            rd_l = pltpu.make_async_remote_copy(
                src_ref=comm.at[ss, pl.ds(m_half, m_half)],
                dst_ref=comm.at[sr, pl.ds(m_half, m_half)],
                send_sem=send_l.at[ss], recv_sem=recv_l.at[sr],
                device_id=(left,), device_id_type=pl.DeviceIdType.MESH,
            )
            rd_r.start()
            rd_l.start()

            @pl.when(h >= 1)
            def _():
                pcp_t, pcp_b = out_copies(
                    lax.rem(my - (h - 1) + N_DEV, N_DEV),
                    lax.rem(my + (h - 1), N_DEV))
                pcp_t.wait()
                pcp_b.wait()

            compute(ss,
                    lax.rem(my - h + N_DEV, N_DEV),
                    lax.rem(my + h, N_DEV))

            rd_r.wait_send()
            rd_l.wait_send()

            @pl.when(h < N_DEV - 2)
            def _():
                pl.semaphore_signal(
                    cap_r, inc=1,
                    device_id=(left,), device_id_type=pl.DeviceIdType.MESH)
                pl.semaphore_signal(
                    cap_l, inc=1,
                    device_id=(right,), device_id_type=pl.DeviceIdType.MESH)

            rd_r.wait_recv()
            rd_l.wait_recv()

            return carry

        lax.fori_loop(0, N_DEV - 1, hop, 0)

        lcp_t, lcp_b = out_copies(
            lax.rem(my - (N_DEV - 2) + N_DEV, N_DEV),
            lax.rem(my + (N_DEV - 2), N_DEV))
        lcp_t.wait()
        lcp_b.wait()
        compute(1,
                lax.rem(my + 1, N_DEV),
                lax.rem(my + N_DEV - 1, N_DEV))
        fcp_t, fcp_b = out_copies(
            lax.rem(my + 1, N_DEV), lax.rem(my + N_DEV - 1, N_DEV))
        fcp_t.wait()
        fcp_b.wait()

    return pl.pallas_call(
        body,
        out_shape=jax.ShapeDtypeStruct((N_DEV * m_per, n), jnp.bfloat16),
        in_specs=[
            pl.BlockSpec(memory_space=pl.ANY),
            pl.BlockSpec(memory_space=pltpu.VMEM),
        ],
        out_specs=pl.BlockSpec(memory_space=pl.ANY),
        scratch_shapes=[
            pltpu.VMEM((2, m_per, k), jnp.bfloat16),
            pltpu.VMEM((m_per, n), jnp.bfloat16),
            pltpu.SemaphoreType.DMA((2,)),
            pltpu.SemaphoreType.DMA((2,)),
            pltpu.SemaphoreType.DMA((2,)),
            pltpu.SemaphoreType.DMA((2,)),
            pltpu.SemaphoreType.DMA((2,)),
            pltpu.SemaphoreType.REGULAR,
            pltpu.SemaphoreType.REGULAR,
        ],
        compiler_params=pltpu.CompilerParams(
            collective_id=0,
            vmem_limit_bytes=100 * 1024 * 1024,
        ),
    )(a16, b16)
